# Initial kernel scaffold; baseline (speedup 1.0000x reference)
import functools

import jax
import jax.numpy as jnp
from jax import lax
from jax.experimental import pallas as pl
from jax.experimental.pallas import tpu as pltpu
from jax.experimental.pallas import tpu_sc as plsc

CH = 128
NUM_GRAPHS = 32
BN = 1000


def _sc_edge_pass1(srcp, dstp, ewp, x, zeros4, n, ep):
    nch = ep // (32 * CH)
    nrows = n // 16
    mesh = plsc.VectorSubcoreMesh(core_axis_name="c", subcore_axis_name="s")

    @functools.partial(
        pl.kernel, mesh=mesh,
        out_type=jax.ShapeDtypeStruct((2 * n, 4), jnp.float32),
        scratch_types=[
            pltpu.VMEM_SHARED((n, 4), jnp.float32),
            pltpu.VMEM((CH,), jnp.int32),
            pltpu.VMEM((CH,), jnp.int32),
            pltpu.VMEM((CH,), jnp.float32),
            pltpu.VMEM((CH, 4), jnp.float32),
            pltpu.SemaphoreType.DMA,
        ])
    def kern(src_h, dst_h, e_h, x_h, z_h, out_h, acc, srcv, dstv, ev, rows, sem):
        c = lax.axis_index("c")
        s = lax.axis_index("s")
        w = s * 2 + c
        r0 = s * nrows
        pltpu.sync_copy(z_h.at[pl.ds(r0, nrows)], acc.at[pl.ds(r0, nrows)])
        plsc.subcore_barrier()
        iota = lax.iota(jnp.int32, 16)
        rq = jnp.right_shift(iota, 2)
        colid = jnp.bitwise_and(iota, 3)

        def body(g, carry):
            base = (w * nch + g) * CH
            pltpu.sync_copy(src_h.at[pl.ds(base, CH)], srcv)
            pltpu.sync_copy(dst_h.at[pl.ds(base, CH)], dstv)
            pltpu.sync_copy(e_h.at[pl.ds(base, CH)], ev)
            pltpu.async_copy(x_h.at[srcv], rows, sem).wait()
            for q in range(CH // 4):
                ridx = rq + (4 * q)
                vals = plsc.load_gather(rows, [ridx, colid])
                er = plsc.load_gather(ev, [ridx])
                plsc.store_scatter(rows, [ridx, colid], vals * er)
            pltpu.sync_copy(rows, acc.at[dstv], add=True)
            return carry

        lax.fori_loop(0, nch, body, 0)
        plsc.subcore_barrier()
        pltpu.sync_copy(acc.at[pl.ds(r0, nrows)],
                        out_h.at[pl.ds(c * n + r0, nrows)])

    return kern(srcp, dstp, ewp, x, zeros4)


def _sc_edge_pass2(srcp, dstp, ewp, ypack, zeros32, n, ep):
    nch = ep // (16 * CH)
    nrows = n // 16
    mesh = plsc.VectorSubcoreMesh(core_axis_name="c", subcore_axis_name="s")

    @functools.partial(
        pl.kernel, mesh=mesh,
        out_type=jax.ShapeDtypeStruct((2 * n, 32), jnp.float32),
        scratch_types=[
            pltpu.VMEM_SHARED((n, 32), jnp.float32),
            pltpu.VMEM((CH,), jnp.int32),
            pltpu.VMEM((CH,), jnp.int32),
            pltpu.VMEM((CH,), jnp.float32),
            pltpu.VMEM((CH,), jnp.int32),
            pltpu.VMEM((CH, 32), jnp.float32),
            pltpu.SemaphoreType.DMA,
        ])
    def kern(src_h, dst_h, e_h, y_h, z_h, out_h,
             acc, srcv, dstv, ev, idx2, rows, sem):
        c = lax.axis_index("c")
        s = lax.axis_index("s")
        r0 = s * nrows
        coff = c * n
        pltpu.sync_copy(z_h.at[pl.ds(r0, nrows)], acc.at[pl.ds(r0, nrows)])
        plsc.subcore_barrier()

        def body(g, carry):
            base = (s * nch + g) * CH
            pltpu.sync_copy(src_h.at[pl.ds(base, CH)], srcv)
            pltpu.sync_copy(dst_h.at[pl.ds(base, CH)], dstv)
            pltpu.sync_copy(e_h.at[pl.ds(base, CH)], ev)
            for q in range(CH // 16):
                idx2[pl.ds(16 * q, 16)] = srcv[pl.ds(16 * q, 16)] + coff
            pltpu.async_copy(y_h.at[idx2], rows, sem).wait()

            def scale(i, cc):
                es = ev[i]
                rows[i, pl.ds(0, 16)] = rows[i, pl.ds(0, 16)] * es
                rows[i, pl.ds(16, 16)] = rows[i, pl.ds(16, 16)] * es
                return cc

            lax.fori_loop(0, CH, scale, 0)
            pltpu.sync_copy(rows, acc.at[dstv], add=True)
            return carry

        lax.fori_loop(0, nch, body, 0)
        plsc.subcore_barrier()
        pltpu.sync_copy(acc.at[pl.ds(r0, nrows)],
                        out_h.at[pl.ds(coff + r0, nrows)])

    return kern(srcp, dstp, ewp, ypack, zeros32)


def _tc_mid(agg1p, x, w1relT, w1rootT, b1, w2relT, w2rootT, n):
    steps = n // BN

    def body(ag_ref, x_ref, wa_ref, wb_ref, b1_ref, w2a_ref, w2b_ref,
             yp_ref, r2_ref):
        a = ag_ref[0] + ag_ref[1]
        x1 = jnp.dot(a, wa_ref[...], preferred_element_type=jnp.float32)
        x1 = x1 + jnp.dot(x_ref[...], wb_ref[...],
                          preferred_element_type=jnp.float32)
        x1 = jnp.maximum(x1 + b1_ref[...], 0.0)
        y = jnp.dot(x1, w2a_ref[...], preferred_element_type=jnp.float32)
        r2_ref[...] = jnp.dot(x1, w2b_ref[...],
                              preferred_element_type=jnp.float32)
        yp_ref[0] = y[:, :32]
        yp_ref[1] = y[:, 32:]

    return pl.pallas_call(
        body,
        grid=(steps,),
        in_specs=[
            pl.BlockSpec((2, BN, 4), lambda i: (0, i, 0)),
            pl.BlockSpec((BN, 4), lambda i: (i, 0)),
            pl.BlockSpec((4, 128), lambda i: (0, 0)),
            pl.BlockSpec((4, 128), lambda i: (0, 0)),
            pl.BlockSpec((1, 128), lambda i: (0, 0)),
            pl.BlockSpec((128, 64), lambda i: (0, 0)),
            pl.BlockSpec((128, 64), lambda i: (0, 0)),
        ],
        out_specs=[
            pl.BlockSpec((2, BN, 32), lambda i: (0, i, 0)),
            pl.BlockSpec((BN, 64), lambda i: (i, 0)),
        ],
        out_shape=[
            jax.ShapeDtypeStruct((2, n, 32), jnp.float32),
            jax.ShapeDtypeStruct((n, 64), jnp.float32),
        ],
    )(agg1p, x, w1relT, w1rootT, b1, w2relT, w2rootT)


def _tc_final(agg2, r2, batchf, b2, wf1T, bf1, wf2T, bf2, n):
    steps = n // BN
    g = NUM_GRAPHS

    def body(ag_ref, r2_ref, b_ref, b2_ref, w1_ref, bf1_ref, w2_ref, bf2_ref,
             out_ref, pooled, cnt):
        i = pl.program_id(0)

        @pl.when(i == 0)
        def _():
            pooled[...] = jnp.zeros_like(pooled)
            cnt[...] = jnp.zeros_like(cnt)

        x2 = jnp.concatenate([ag_ref[0], ag_ref[1]], axis=1)
        x2 = jnp.maximum(x2 + r2_ref[...] + b2_ref[...], 0.0)
        bf = b_ref[0]
        oh = (bf == lax.broadcasted_iota(jnp.float32, (BN, g), 1))
        oh = oh.astype(jnp.float32)
        pooled[...] += lax.dot_general(
            oh, x2, (((0,), (0,)), ((), ())),
            preferred_element_type=jnp.float32)
        cnt[...] += lax.dot_general(
            oh, jnp.ones((BN, 64), jnp.float32), (((0,), (0,)), ((), ())),
            preferred_element_type=jnp.float32)

        @pl.when(i == steps - 1)
        def _():
            m = pooled[...] / jnp.maximum(cnt[...], 1.0)
            h1 = jnp.dot(m, w1_ref[...], preferred_element_type=jnp.float32)
            h1 = jnp.maximum(h1 + bf1_ref[...], 0.0)
            out_ref[...] = (jnp.dot(h1, w2_ref[...],
                                    preferred_element_type=jnp.float32)
                            + bf2_ref[...])

    return pl.pallas_call(
        body,
        grid=(steps,),
        in_specs=[
            pl.BlockSpec((2, BN, 32), lambda i: (0, i, 0)),
            pl.BlockSpec((BN, 64), lambda i: (i, 0)),
            pl.BlockSpec((1, BN, 1), lambda i: (i, 0, 0)),
            pl.BlockSpec((1, 64), lambda i: (0, 0)),
            pl.BlockSpec((64, 32), lambda i: (0, 0)),
            pl.BlockSpec((1, 32), lambda i: (0, 0)),
            pl.BlockSpec((32, 3), lambda i: (0, 0)),
            pl.BlockSpec((1, 3), lambda i: (0, 0)),
        ],
        out_specs=pl.BlockSpec((g, 3), lambda i: (0, 0)),
        out_shape=jax.ShapeDtypeStruct((g, 3), jnp.float32),
        scratch_shapes=[
            pltpu.VMEM((g, 64), jnp.float32),
            pltpu.VMEM((g, 64), jnp.float32),
        ],
    )(agg2, r2, batchf, b2, wf1T, bf1, wf2T, bf2)


def kernel(pos, z, edge_index, edge_attr, batch, W1_rel, b1_rel, W1_root,
           W2_rel, b2_rel, W2_root, Wfc1, bfc1, Wfc2, bfc2):
    n = pos.shape[0]
    e = edge_attr.shape[0]
    epad = ((e + 32 * CH - 1) // (32 * CH)) * (32 * CH)
    pad = epad - e
    src = jnp.concatenate([edge_index[0], jnp.zeros((pad,), jnp.int32)])
    dst = jnp.concatenate([edge_index[1], jnp.zeros((pad,), jnp.int32)])
    ew = jnp.concatenate([edge_attr, jnp.zeros((pad,), jnp.float32)])

    x = jnp.concatenate([pos, z[:, None]], axis=1)
    zeros4 = jnp.zeros((n, 4), jnp.float32)
    zeros32 = jnp.zeros((n, 32), jnp.float32)

    agg1p = _sc_edge_pass1(src, dst, ew, x, zeros4, n, epad)
    agg1p = agg1p.reshape(2, n, 4)

    ypack, r2 = _tc_mid(agg1p, x, W1_rel.T, W1_root.T, b1_rel[None, :],
                        W2_rel.T, W2_root.T, n)

    agg2 = _sc_edge_pass2(src, dst, ew, ypack.reshape(2 * n, 32), zeros32,
                          n, epad)
    agg2 = agg2.reshape(2, n, 32)

    steps = n // BN
    batchf = batch.astype(jnp.float32).reshape(steps, BN, 1)
    out = _tc_final(agg2, r2, batchf, b2_rel[None, :], Wfc1.T, bfc1[None, :],
                    Wfc2.T, bfc2[None, :], n)
    return out


# speedup vs baseline: 4.0639x; 4.0639x over previous
import functools

import jax
import jax.numpy as jnp
from jax import lax
from jax.experimental import pallas as pl
from jax.experimental.pallas import tpu as pltpu
from jax.experimental.pallas import tpu_sc as plsc

CH = 128
NUM_GRAPHS = 32
BN = 1000


def _sc_edge_pass1(srcp, dstp, ewp, x16, zeros16, n, npad, ep):
    nch = ep // (32 * CH)
    nrows = npad // 16
    tail = n - 15 * nrows
    mesh = plsc.VectorSubcoreMesh(core_axis_name="c", subcore_axis_name="s")

    @functools.partial(
        pl.kernel, mesh=mesh,
        compiler_params=pltpu.CompilerParams(use_tc_tiling_on_sc=False),
        out_type=jax.ShapeDtypeStruct((2 * n, 16), jnp.float32),
        scratch_types=[
            pltpu.VMEM_SHARED((npad, 16), jnp.float32),
            pltpu.VMEM((CH,), jnp.int32),
            pltpu.VMEM((CH,), jnp.int32),
            pltpu.VMEM((CH + 16,), jnp.float32),
            pltpu.VMEM((CH, 16), jnp.float32),
            pltpu.SemaphoreType.DMA,
        ])
    def kern(src_h, dst_h, e_h, x_h, z_h, out_h, acc, srcv, dstv, ev, rows, sem):
        c = lax.axis_index("c")
        s = lax.axis_index("s")
        w = s * 2 + c
        r0 = s * nrows
        pltpu.sync_copy(z_h.at[pl.ds(r0, nrows)], acc.at[pl.ds(r0, nrows)])
        plsc.subcore_barrier()

        def body(g, carry):
            base = (w * nch + g) * CH
            pltpu.sync_copy(src_h.at[pl.ds(base, CH)], srcv)
            pltpu.sync_copy(dst_h.at[pl.ds(base, CH)], dstv)
            pltpu.sync_copy(e_h.at[pl.ds(base, CH)], ev.at[pl.ds(0, CH)])
            pltpu.async_copy(x_h.at[srcv], rows, sem).wait()

            def scale(i, cc):
                es = ev[pl.ds(i, 16)][0]
                rows[i] = rows[i] * es
                return cc

            lax.fori_loop(0, CH, scale, 0)
            pltpu.sync_copy(rows, acc.at[dstv], add=True)
            return carry

        lax.fori_loop(0, nch, body, 0)
        plsc.subcore_barrier()

        @pl.when(s < 15)
        def _():
            pltpu.sync_copy(acc.at[pl.ds(r0, nrows)],
                            out_h.at[pl.ds(c * n + r0, nrows)])

        @pl.when(s == 15)
        def _():
            pltpu.sync_copy(acc.at[pl.ds(r0, tail)],
                            out_h.at[pl.ds(c * n + r0, tail)])

    return kern(srcp, dstp, ewp, x16, zeros16)


def _sc_edge_pass2(srcp, dstp, ewp, ypack, zeros32, n, npad, ep):
    nch = ep // (16 * CH)
    nrows = npad // 16
    tail = n - 15 * nrows
    mesh = plsc.VectorSubcoreMesh(core_axis_name="c", subcore_axis_name="s")

    @functools.partial(
        pl.kernel, mesh=mesh,
        compiler_params=pltpu.CompilerParams(use_tc_tiling_on_sc=False),
        out_type=jax.ShapeDtypeStruct((2 * n, 32), jnp.float32),
        scratch_types=[
            pltpu.VMEM_SHARED((npad, 32), jnp.float32),
            pltpu.VMEM((CH,), jnp.int32),
            pltpu.VMEM((CH,), jnp.int32),
            pltpu.VMEM((CH + 16,), jnp.float32),
            pltpu.VMEM((CH,), jnp.int32),
            pltpu.VMEM((CH, 32), jnp.float32),
            pltpu.SemaphoreType.DMA,
        ])
    def kern(src_h, dst_h, e_h, y_h, z_h, out_h,
             acc, srcv, dstv, ev, idx2, rows, sem):
        c = lax.axis_index("c")
        s = lax.axis_index("s")
        r0 = s * nrows
        coff = c * n
        pltpu.sync_copy(z_h.at[pl.ds(r0, nrows)], acc.at[pl.ds(r0, nrows)])
        plsc.subcore_barrier()

        def body(g, carry):
            base = (s * nch + g) * CH
            pltpu.sync_copy(src_h.at[pl.ds(base, CH)], srcv)
            pltpu.sync_copy(dst_h.at[pl.ds(base, CH)], dstv)
            pltpu.sync_copy(e_h.at[pl.ds(base, CH)], ev.at[pl.ds(0, CH)])
            for q in range(CH // 16):
                idx2[pl.ds(16 * q, 16)] = srcv[pl.ds(16 * q, 16)] + coff
            pltpu.async_copy(y_h.at[idx2], rows, sem).wait()

            def scale(i, cc):
                es = ev[pl.ds(i, 16)][0]
                rows[i, pl.ds(0, 16)] = rows[i, pl.ds(0, 16)] * es
                rows[i, pl.ds(16, 16)] = rows[i, pl.ds(16, 16)] * es
                return cc

            lax.fori_loop(0, CH, scale, 0)
            pltpu.sync_copy(rows, acc.at[dstv], add=True)
            return carry

        lax.fori_loop(0, nch, body, 0)
        plsc.subcore_barrier()

        @pl.when(s < 15)
        def _():
            pltpu.sync_copy(acc.at[pl.ds(r0, nrows)],
                            out_h.at[pl.ds(coff + r0, nrows)])

        @pl.when(s == 15)
        def _():
            pltpu.sync_copy(acc.at[pl.ds(r0, tail)],
                            out_h.at[pl.ds(coff + r0, tail)])

    return kern(srcp, dstp, ewp, ypack, zeros32)


def _tc_mid(agg1p, x, w1relT, w1rootT, b1, w2relT, w2rootT, n):
    steps = n // BN

    def body(ag_ref, x_ref, wa_ref, wb_ref, b1_ref, w2a_ref, w2b_ref,
             yp_ref, r2_ref):
        a = ag_ref[0, :, :4] + ag_ref[1, :, :4]
        x1 = jnp.dot(a, wa_ref[...], preferred_element_type=jnp.float32)
        x1 = x1 + jnp.dot(x_ref[...], wb_ref[...],
                          preferred_element_type=jnp.float32)
        x1 = jnp.maximum(x1 + b1_ref[...], 0.0)
        y = jnp.dot(x1, w2a_ref[...], preferred_element_type=jnp.float32)
        r2_ref[...] = jnp.dot(x1, w2b_ref[...],
                              preferred_element_type=jnp.float32)
        yp_ref[0] = y[:, :32]
        yp_ref[1] = y[:, 32:]

    return pl.pallas_call(
        body,
        grid=(steps,),
        in_specs=[
            pl.BlockSpec((2, BN, 16), lambda i: (0, i, 0)),
            pl.BlockSpec((BN, 4), lambda i: (i, 0)),
            pl.BlockSpec((4, 128), lambda i: (0, 0)),
            pl.BlockSpec((4, 128), lambda i: (0, 0)),
            pl.BlockSpec((1, 128), lambda i: (0, 0)),
            pl.BlockSpec((128, 64), lambda i: (0, 0)),
            pl.BlockSpec((128, 64), lambda i: (0, 0)),
        ],
        out_specs=[
            pl.BlockSpec((2, BN, 32), lambda i: (0, i, 0)),
            pl.BlockSpec((BN, 64), lambda i: (i, 0)),
        ],
        out_shape=[
            jax.ShapeDtypeStruct((2, n, 32), jnp.float32),
            jax.ShapeDtypeStruct((n, 64), jnp.float32),
        ],
    )(agg1p, x, w1relT, w1rootT, b1, w2relT, w2rootT)


def _tc_final(agg2, r2, batchf, b2, wf1T, bf1, wf2T, bf2, n):
    steps = n // BN
    g = NUM_GRAPHS

    def body(ag_ref, r2_ref, b_ref, b2_ref, w1_ref, bf1_ref, w2_ref, bf2_ref,
             out_ref, pooled, cnt):
        i = pl.program_id(0)

        @pl.when(i == 0)
        def _():
            pooled[...] = jnp.zeros_like(pooled)
            cnt[...] = jnp.zeros_like(cnt)

        x2 = jnp.concatenate([ag_ref[0], ag_ref[1]], axis=1)
        x2 = jnp.maximum(x2 + r2_ref[...] + b2_ref[...], 0.0)
        bf = b_ref[0]
        gid = lax.broadcasted_iota(jnp.int32, (BN, g), 1).astype(jnp.float32)
        oh = (bf == gid)
        oh = oh.astype(jnp.float32)
        pooled[...] += lax.dot_general(
            oh, x2, (((0,), (0,)), ((), ())),
            preferred_element_type=jnp.float32)
        cnt[...] += lax.dot_general(
            oh, jnp.ones((BN, 64), jnp.float32), (((0,), (0,)), ((), ())),
            preferred_element_type=jnp.float32)

        @pl.when(i == steps - 1)
        def _():
            m = pooled[...] / jnp.maximum(cnt[...], 1.0)
            h1 = jnp.dot(m, w1_ref[...], preferred_element_type=jnp.float32)
            h1 = jnp.maximum(h1 + bf1_ref[...], 0.0)
            out_ref[...] = (jnp.dot(h1, w2_ref[...],
                                    preferred_element_type=jnp.float32)
                            + bf2_ref[...])

    return pl.pallas_call(
        body,
        grid=(steps,),
        in_specs=[
            pl.BlockSpec((2, BN, 32), lambda i: (0, i, 0)),
            pl.BlockSpec((BN, 64), lambda i: (i, 0)),
            pl.BlockSpec((1, BN, 1), lambda i: (i, 0, 0)),
            pl.BlockSpec((1, 64), lambda i: (0, 0)),
            pl.BlockSpec((64, 32), lambda i: (0, 0)),
            pl.BlockSpec((1, 32), lambda i: (0, 0)),
            pl.BlockSpec((32, 3), lambda i: (0, 0)),
            pl.BlockSpec((1, 3), lambda i: (0, 0)),
        ],
        out_specs=pl.BlockSpec((g, 3), lambda i: (0, 0)),
        out_shape=jax.ShapeDtypeStruct((g, 3), jnp.float32),
        scratch_shapes=[
            pltpu.VMEM((g, 64), jnp.float32),
            pltpu.VMEM((g, 64), jnp.float32),
        ],
    )(agg2, r2, batchf, b2, wf1T, bf1, wf2T, bf2)


def kernel(pos, z, edge_index, edge_attr, batch, W1_rel, b1_rel, W1_root,
           W2_rel, b2_rel, W2_root, Wfc1, bfc1, Wfc2, bfc2):
    n = pos.shape[0]
    e = edge_attr.shape[0]
    epad = ((e + 32 * CH - 1) // (32 * CH)) * (32 * CH)
    pad = epad - e
    src = jnp.concatenate([edge_index[0], jnp.zeros((pad,), jnp.int32)])
    dst = jnp.concatenate([edge_index[1], jnp.zeros((pad,), jnp.int32)])
    ew = jnp.concatenate([edge_attr, jnp.zeros((pad,), jnp.float32)])

    x = jnp.concatenate([pos, z[:, None]], axis=1)
    x16 = jnp.concatenate([x, jnp.zeros((n, 12), jnp.float32)], axis=1)
    npad = ((n + 127) // 128) * 128
    zeros16 = jnp.zeros((npad, 16), jnp.float32)
    zeros32 = jnp.zeros((npad, 32), jnp.float32)

    agg1p = _sc_edge_pass1(src, dst, ew, x16, zeros16, n, npad, epad)
    agg1p = agg1p.reshape(2, n, 16)

    ypack, r2 = _tc_mid(agg1p, x, W1_rel.T, W1_root.T, b1_rel[None, :],
                        W2_rel.T, W2_root.T, n)

    agg2 = _sc_edge_pass2(src, dst, ew, ypack.reshape(2 * n, 32), zeros32,
                          n, npad, epad)
    agg2 = agg2.reshape(2, n, 32)

    steps = n // BN
    batchf = batch.astype(jnp.float32).reshape(steps, BN, 1)
    out = _tc_final(agg2, r2, batchf, b2_rel[None, :], Wfc1.T, bfc1[None, :],
                    Wfc2.T, bfc2[None, :], n)
    return out


# speedup vs baseline: 4.8410x; 1.1912x over previous
import functools

import jax
import jax.numpy as jnp
from jax import lax
from jax.experimental import pallas as pl
from jax.experimental.pallas import tpu as pltpu
from jax.experimental.pallas import tpu_sc as plsc

CH = 128
NUM_GRAPHS = 32
BN = 1000


def _sc_edge_pass1(srcp, dstp, ewp, x16, zeros16, n, npad, ep):
    nch = ep // (32 * CH)
    nrows = npad // 16
    tail = n - 15 * nrows
    mesh = plsc.VectorSubcoreMesh(core_axis_name="c", subcore_axis_name="s")

    @functools.partial(
        pl.kernel, mesh=mesh,
        compiler_params=pltpu.CompilerParams(use_tc_tiling_on_sc=False),
        out_type=jax.ShapeDtypeStruct((2 * n, 16), jnp.float32),
        scratch_types=[
            pltpu.VMEM_SHARED((npad, 16), jnp.float32),
            pltpu.VMEM((CH,), jnp.int32),
            pltpu.VMEM((CH,), jnp.int32),
            pltpu.VMEM((CH,), jnp.float32),
            pltpu.VMEM((CH, 16), jnp.float32),
            pltpu.SemaphoreType.DMA,
        ])
    def kern(src_h, dst_h, e_h, x_h, z_h, out_h, acc, srcv, dstv, ev, rows, sem):
        c = lax.axis_index("c")
        s = lax.axis_index("s")
        w = s * 2 + c
        r0 = s * nrows
        pltpu.sync_copy(z_h.at[pl.ds(r0, nrows)], acc.at[pl.ds(r0, nrows)])
        plsc.subcore_barrier()

        def body(g, carry):
            base = (w * nch + g) * CH
            pltpu.sync_copy(src_h.at[pl.ds(base, CH)], srcv)
            pltpu.sync_copy(dst_h.at[pl.ds(base, CH)], dstv)
            pltpu.sync_copy(e_h.at[pl.ds(base, CH)], ev)
            pltpu.async_copy(x_h.at[srcv], rows, sem).wait()

            def scale(q, cc):
                evec = ev[pl.ds(16 * q, 16)]
                for j in range(16):
                    r = 16 * q + j
                    rows[r] = rows[r] * evec[j]
                return cc

            lax.fori_loop(0, CH // 16, scale, 0)
            pltpu.sync_copy(rows, acc.at[dstv], add=True)
            return carry

        lax.fori_loop(0, nch, body, 0)
        plsc.subcore_barrier()

        @pl.when(s < 15)
        def _():
            pltpu.sync_copy(acc.at[pl.ds(r0, nrows)],
                            out_h.at[pl.ds(c * n + r0, nrows)])

        @pl.when(s == 15)
        def _():
            pltpu.sync_copy(acc.at[pl.ds(r0, tail)],
                            out_h.at[pl.ds(c * n + r0, tail)])

    return kern(srcp, dstp, ewp, x16, zeros16)


def _sc_edge_pass2(srcp, dstp, ewp, ypack, zeros32, n, npad, ep):
    nch = ep // (16 * CH)
    nrows = npad // 16
    tail = n - 15 * nrows
    mesh = plsc.VectorSubcoreMesh(core_axis_name="c", subcore_axis_name="s")

    @functools.partial(
        pl.kernel, mesh=mesh,
        compiler_params=pltpu.CompilerParams(use_tc_tiling_on_sc=False),
        out_type=jax.ShapeDtypeStruct((2 * n, 32), jnp.float32),
        scratch_types=[
            pltpu.VMEM_SHARED((npad, 32), jnp.float32),
            pltpu.VMEM((CH,), jnp.int32),
            pltpu.VMEM((CH,), jnp.int32),
            pltpu.VMEM((CH,), jnp.float32),
            pltpu.VMEM((CH,), jnp.int32),
            pltpu.VMEM((CH, 32), jnp.float32),
            pltpu.SemaphoreType.DMA,
        ])
    def kern(src_h, dst_h, e_h, y_h, z_h, out_h,
             acc, srcv, dstv, ev, idx2, rows, sem):
        c = lax.axis_index("c")
        s = lax.axis_index("s")
        r0 = s * nrows
        coff = c * n
        pltpu.sync_copy(z_h.at[pl.ds(r0, nrows)], acc.at[pl.ds(r0, nrows)])
        plsc.subcore_barrier()

        def body(g, carry):
            base = (s * nch + g) * CH
            pltpu.sync_copy(src_h.at[pl.ds(base, CH)], srcv)
            pltpu.sync_copy(dst_h.at[pl.ds(base, CH)], dstv)
            pltpu.sync_copy(e_h.at[pl.ds(base, CH)], ev)
            for q in range(CH // 16):
                idx2[pl.ds(16 * q, 16)] = srcv[pl.ds(16 * q, 16)] + coff
            pltpu.async_copy(y_h.at[idx2], rows, sem).wait()

            def scale(q, cc):
                evec = ev[pl.ds(16 * q, 16)]
                for j in range(16):
                    r = 16 * q + j
                    rows[r, pl.ds(0, 16)] = rows[r, pl.ds(0, 16)] * evec[j]
                    rows[r, pl.ds(16, 16)] = rows[r, pl.ds(16, 16)] * evec[j]
                return cc

            lax.fori_loop(0, CH // 16, scale, 0)
            pltpu.sync_copy(rows, acc.at[dstv], add=True)
            return carry

        lax.fori_loop(0, nch, body, 0)
        plsc.subcore_barrier()

        @pl.when(s < 15)
        def _():
            pltpu.sync_copy(acc.at[pl.ds(r0, nrows)],
                            out_h.at[pl.ds(coff + r0, nrows)])

        @pl.when(s == 15)
        def _():
            pltpu.sync_copy(acc.at[pl.ds(r0, tail)],
                            out_h.at[pl.ds(coff + r0, tail)])

    return kern(srcp, dstp, ewp, ypack, zeros32)


def _tc_mid(agg1p, x, w1relT, w1rootT, b1, w2relT, w2rootT, n):
    steps = n // BN

    def body(ag_ref, x_ref, wa_ref, wb_ref, b1_ref, w2a_ref, w2b_ref,
             yp_ref, r2_ref):
        a = ag_ref[0, :, :4] + ag_ref[1, :, :4]
        x1 = jnp.dot(a, wa_ref[...], preferred_element_type=jnp.float32)
        x1 = x1 + jnp.dot(x_ref[...], wb_ref[...],
                          preferred_element_type=jnp.float32)
        x1 = jnp.maximum(x1 + b1_ref[...], 0.0)
        y = jnp.dot(x1, w2a_ref[...], preferred_element_type=jnp.float32)
        r2_ref[...] = jnp.dot(x1, w2b_ref[...],
                              preferred_element_type=jnp.float32)
        yp_ref[0] = y[:, :32]
        yp_ref[1] = y[:, 32:]

    return pl.pallas_call(
        body,
        grid=(steps,),
        in_specs=[
            pl.BlockSpec((2, BN, 16), lambda i: (0, i, 0)),
            pl.BlockSpec((BN, 4), lambda i: (i, 0)),
            pl.BlockSpec((4, 128), lambda i: (0, 0)),
            pl.BlockSpec((4, 128), lambda i: (0, 0)),
            pl.BlockSpec((1, 128), lambda i: (0, 0)),
            pl.BlockSpec((128, 64), lambda i: (0, 0)),
            pl.BlockSpec((128, 64), lambda i: (0, 0)),
        ],
        out_specs=[
            pl.BlockSpec((2, BN, 32), lambda i: (0, i, 0)),
            pl.BlockSpec((BN, 64), lambda i: (i, 0)),
        ],
        out_shape=[
            jax.ShapeDtypeStruct((2, n, 32), jnp.float32),
            jax.ShapeDtypeStruct((n, 64), jnp.float32),
        ],
    )(agg1p, x, w1relT, w1rootT, b1, w2relT, w2rootT)


def _tc_final(agg2, r2, batchf, b2, wf1T, bf1, wf2T, bf2, n):
    steps = n // BN
    g = NUM_GRAPHS

    def body(ag_ref, r2_ref, b_ref, b2_ref, w1_ref, bf1_ref, w2_ref, bf2_ref,
             out_ref, pooled, cnt):
        i = pl.program_id(0)

        @pl.when(i == 0)
        def _():
            pooled[...] = jnp.zeros_like(pooled)
            cnt[...] = jnp.zeros_like(cnt)

        x2 = jnp.concatenate([ag_ref[0], ag_ref[1]], axis=1)
        x2 = jnp.maximum(x2 + r2_ref[...] + b2_ref[...], 0.0)
        bf = b_ref[0]
        gid = lax.broadcasted_iota(jnp.int32, (BN, g), 1).astype(jnp.float32)
        oh = (bf == gid)
        oh = oh.astype(jnp.float32)
        pooled[...] += lax.dot_general(
            oh, x2, (((0,), (0,)), ((), ())),
            preferred_element_type=jnp.float32)
        cnt[...] += lax.dot_general(
            oh, jnp.ones((BN, 64), jnp.float32), (((0,), (0,)), ((), ())),
            preferred_element_type=jnp.float32)

        @pl.when(i == steps - 1)
        def _():
            m = pooled[...] / jnp.maximum(cnt[...], 1.0)
            h1 = jnp.dot(m, w1_ref[...], preferred_element_type=jnp.float32)
            h1 = jnp.maximum(h1 + bf1_ref[...], 0.0)
            out_ref[...] = (jnp.dot(h1, w2_ref[...],
                                    preferred_element_type=jnp.float32)
                            + bf2_ref[...])

    return pl.pallas_call(
        body,
        grid=(steps,),
        in_specs=[
            pl.BlockSpec((2, BN, 32), lambda i: (0, i, 0)),
            pl.BlockSpec((BN, 64), lambda i: (i, 0)),
            pl.BlockSpec((1, BN, 1), lambda i: (i, 0, 0)),
            pl.BlockSpec((1, 64), lambda i: (0, 0)),
            pl.BlockSpec((64, 32), lambda i: (0, 0)),
            pl.BlockSpec((1, 32), lambda i: (0, 0)),
            pl.BlockSpec((32, 3), lambda i: (0, 0)),
            pl.BlockSpec((1, 3), lambda i: (0, 0)),
        ],
        out_specs=pl.BlockSpec((g, 3), lambda i: (0, 0)),
        out_shape=jax.ShapeDtypeStruct((g, 3), jnp.float32),
        scratch_shapes=[
            pltpu.VMEM((g, 64), jnp.float32),
            pltpu.VMEM((g, 64), jnp.float32),
        ],
    )(agg2, r2, batchf, b2, wf1T, bf1, wf2T, bf2)


def kernel(pos, z, edge_index, edge_attr, batch, W1_rel, b1_rel, W1_root,
           W2_rel, b2_rel, W2_root, Wfc1, bfc1, Wfc2, bfc2):
    n = pos.shape[0]
    e = edge_attr.shape[0]
    epad = ((e + 32 * CH - 1) // (32 * CH)) * (32 * CH)
    pad = epad - e
    src = jnp.concatenate([edge_index[0], jnp.zeros((pad,), jnp.int32)])
    dst = jnp.concatenate([edge_index[1], jnp.zeros((pad,), jnp.int32)])
    ew = jnp.concatenate([edge_attr, jnp.zeros((pad,), jnp.float32)])

    x = jnp.concatenate([pos, z[:, None]], axis=1)
    x16 = jnp.concatenate([x, jnp.zeros((n, 12), jnp.float32)], axis=1)
    npad = ((n + 127) // 128) * 128
    zeros16 = jnp.zeros((npad, 16), jnp.float32)
    zeros32 = jnp.zeros((npad, 32), jnp.float32)

    agg1p = _sc_edge_pass1(src, dst, ew, x16, zeros16, n, npad, epad)
    agg1p = agg1p.reshape(2, n, 16)

    ypack, r2 = _tc_mid(agg1p, x, W1_rel.T, W1_root.T, b1_rel[None, :],
                        W2_rel.T, W2_root.T, n)

    agg2 = _sc_edge_pass2(src, dst, ew, ypack.reshape(2 * n, 32), zeros32,
                          n, npad, epad)
    agg2 = agg2.reshape(2, n, 32)

    steps = n // BN
    batchf = batch.astype(jnp.float32).reshape(steps, BN, 1)
    out = _tc_final(agg2, r2, batchf, b2_rel[None, :], Wfc1.T, bfc1[None, :],
                    Wfc2.T, bfc2[None, :], n)
    return out


# speedup vs baseline: 10.6088x; 2.1915x over previous
import functools

import jax
import jax.numpy as jnp
from jax import lax
from jax.experimental import pallas as pl
from jax.experimental.pallas import tpu as pltpu
from jax.experimental.pallas import tpu_sc as plsc

SUB = 128
K = 4
KCH = K * SUB
NUM_GRAPHS = 32
BN = 1000


def _scale_rows16(rows, ev, ngroups):
    def scale(q, cc):
        evec = ev[pl.ds(16 * q, 16)]
        for j in range(16):
            r = 16 * q + j
            rows[r] = rows[r] * evec[j]
        return cc

    lax.fori_loop(0, ngroups, scale, 0)


def _scale_rows32(rows, ev, ngroups):
    def scale(q, cc):
        evec = ev[pl.ds(16 * q, 16)]
        for j in range(16):
            r = 16 * q + j
            rows[r, pl.ds(0, 16)] = rows[r, pl.ds(0, 16)] * evec[j]
            rows[r, pl.ds(16, 16)] = rows[r, pl.ds(16, 16)] * evec[j]
        return cc

    lax.fori_loop(0, ngroups, scale, 0)


def _sc_edge_pass1(src2, dst2, ewp, x16, zeros16, n, npad, ep):
    nsk = ep // (32 * KCH)
    nrows = npad // 16
    tail = n - 15 * nrows
    mesh = plsc.VectorSubcoreMesh(core_axis_name="c", subcore_axis_name="s")

    @functools.partial(
        pl.kernel, mesh=mesh,
        compiler_params=pltpu.CompilerParams(use_tc_tiling_on_sc=False),
        out_type=jax.ShapeDtypeStruct((2 * n, 16), jnp.float32),
        scratch_types=[
            pltpu.VMEM_SHARED((npad, 16), jnp.float32),
            pltpu.VMEM((K, SUB), jnp.int32),
            pltpu.VMEM((K, SUB), jnp.int32),
            pltpu.VMEM((KCH,), jnp.float32),
            pltpu.VMEM((KCH, 16), jnp.float32),
            pltpu.SemaphoreType.DMA,
            pltpu.SemaphoreType.DMA,
            pltpu.SemaphoreType.DMA,
        ])
    def kern(src_h, dst_h, e_h, x_h, z_h, out_h,
             acc, srcv, dstv, ev, rows, sem_e, sem_g, sem_s):
        c = lax.axis_index("c")
        s = lax.axis_index("s")
        w = s * 2 + c
        r0 = s * nrows
        pltpu.sync_copy(z_h.at[pl.ds(r0, nrows)], acc.at[pl.ds(r0, nrows)])
        plsc.subcore_barrier()

        def body(g, carry):
            base = (w * nsk + g) * KCH
            cb = (w * nsk + g) * K
            h1 = pltpu.async_copy(src_h.at[pl.ds(cb, K)], srcv, sem_e)
            h2 = pltpu.async_copy(dst_h.at[pl.ds(cb, K)], dstv, sem_e)
            h3 = pltpu.async_copy(e_h.at[pl.ds(base, KCH)], ev, sem_e)
            h1.wait()
            h2.wait()
            h3.wait()
            ghs = [pltpu.async_copy(x_h.at[srcv.at[j]],
                                    rows.at[pl.ds(SUB * j, SUB)], sem_g)
                   for j in range(K)]
            for h in ghs:
                h.wait()
            _scale_rows16(rows, ev, KCH // 16)
            shs = [pltpu.async_copy(rows.at[pl.ds(SUB * j, SUB)],
                                    acc.at[dstv.at[j]], sem_s, add=True)
                   for j in range(K)]
            for h in shs:
                h.wait()
            return carry

        lax.fori_loop(0, nsk, body, 0)
        plsc.subcore_barrier()

        @pl.when(s < 15)
        def _():
            pltpu.sync_copy(acc.at[pl.ds(r0, nrows)],
                            out_h.at[pl.ds(c * n + r0, nrows)])

        @pl.when(s == 15)
        def _():
            pltpu.sync_copy(acc.at[pl.ds(r0, tail)],
                            out_h.at[pl.ds(c * n + r0, tail)])

    return kern(src2, dst2, ewp, x16, zeros16)


def _sc_edge_pass2(src2, dst2, ewp, ypack, zeros32, n, npad, ep):
    nsk = ep // (16 * KCH)
    nrows = npad // 16
    tail = n - 15 * nrows
    mesh = plsc.VectorSubcoreMesh(core_axis_name="c", subcore_axis_name="s")

    @functools.partial(
        pl.kernel, mesh=mesh,
        compiler_params=pltpu.CompilerParams(use_tc_tiling_on_sc=False),
        out_type=jax.ShapeDtypeStruct((2 * n, 32), jnp.float32),
        scratch_types=[
            pltpu.VMEM_SHARED((npad, 32), jnp.float32),
            pltpu.VMEM((K, SUB), jnp.int32),
            pltpu.VMEM((K, SUB), jnp.int32),
            pltpu.VMEM((KCH,), jnp.float32),
            pltpu.VMEM((KCH,), jnp.int32),
            pltpu.VMEM((KCH, 32), jnp.float32),
            pltpu.SemaphoreType.DMA,
            pltpu.SemaphoreType.DMA,
            pltpu.SemaphoreType.DMA,
        ])
    def kern(src_h, dst_h, e_h, y_h, z_h, out_h,
             acc, srcv, dstv, ev, idx2, rows, sem_e, sem_g, sem_s):
        c = lax.axis_index("c")
        s = lax.axis_index("s")
        r0 = s * nrows
        coff = c * n
        pltpu.sync_copy(z_h.at[pl.ds(r0, nrows)], acc.at[pl.ds(r0, nrows)])
        plsc.subcore_barrier()

        def body(g, carry):
            base = (s * nsk + g) * KCH
            cb = (s * nsk + g) * K
            h1 = pltpu.async_copy(src_h.at[pl.ds(cb, K)], srcv, sem_e)
            h2 = pltpu.async_copy(dst_h.at[pl.ds(cb, K)], dstv, sem_e)
            h3 = pltpu.async_copy(e_h.at[pl.ds(base, KCH)], ev, sem_e)
            h1.wait()
            h2.wait()
            h3.wait()
            for j in range(K):
                for m in range(SUB // 16):
                    idx2[pl.ds(SUB * j + 16 * m, 16)] = (
                        srcv[j, pl.ds(16 * m, 16)] + coff)
            ghs = [pltpu.async_copy(y_h.at[idx2.at[pl.ds(SUB * j, SUB)]],
                                    rows.at[pl.ds(SUB * j, SUB)], sem_g)
                   for j in range(K)]
            for h in ghs:
                h.wait()
            _scale_rows32(rows, ev, KCH // 16)
            shs = [pltpu.async_copy(rows.at[pl.ds(SUB * j, SUB)],
                                    acc.at[dstv.at[j]], sem_s, add=True)
                   for j in range(K)]
            for h in shs:
                h.wait()
            return carry

        lax.fori_loop(0, nsk, body, 0)
        plsc.subcore_barrier()

        @pl.when(s < 15)
        def _():
            pltpu.sync_copy(acc.at[pl.ds(r0, nrows)],
                            out_h.at[pl.ds(coff + r0, nrows)])

        @pl.when(s == 15)
        def _():
            pltpu.sync_copy(acc.at[pl.ds(r0, tail)],
                            out_h.at[pl.ds(coff + r0, tail)])

    return kern(src2, dst2, ewp, ypack, zeros32)


def _tc_mid(agg1p, x, w1relT, w1rootT, b1, w2relT, w2rootT, n):
    steps = n // BN

    def body(ag_ref, x_ref, wa_ref, wb_ref, b1_ref, w2a_ref, w2b_ref,
             yp_ref, r2_ref):
        a = ag_ref[0, :, :4] + ag_ref[1, :, :4]
        x1 = jnp.dot(a, wa_ref[...], preferred_element_type=jnp.float32)
        x1 = x1 + jnp.dot(x_ref[...], wb_ref[...],
                          preferred_element_type=jnp.float32)
        x1 = jnp.maximum(x1 + b1_ref[...], 0.0)
        y = jnp.dot(x1, w2a_ref[...], preferred_element_type=jnp.float32)
        r2_ref[...] = jnp.dot(x1, w2b_ref[...],
                              preferred_element_type=jnp.float32)
        yp_ref[0] = y[:, :32]
        yp_ref[1] = y[:, 32:]

    return pl.pallas_call(
        body,
        grid=(steps,),
        in_specs=[
            pl.BlockSpec((2, BN, 16), lambda i: (0, i, 0)),
            pl.BlockSpec((BN, 4), lambda i: (i, 0)),
            pl.BlockSpec((4, 128), lambda i: (0, 0)),
            pl.BlockSpec((4, 128), lambda i: (0, 0)),
            pl.BlockSpec((1, 128), lambda i: (0, 0)),
            pl.BlockSpec((128, 64), lambda i: (0, 0)),
            pl.BlockSpec((128, 64), lambda i: (0, 0)),
        ],
        out_specs=[
            pl.BlockSpec((2, BN, 32), lambda i: (0, i, 0)),
            pl.BlockSpec((BN, 64), lambda i: (i, 0)),
        ],
        out_shape=[
            jax.ShapeDtypeStruct((2, n, 32), jnp.float32),
            jax.ShapeDtypeStruct((n, 64), jnp.float32),
        ],
    )(agg1p, x, w1relT, w1rootT, b1, w2relT, w2rootT)


def _tc_final(agg2, r2, batchf, b2, wf1T, bf1, wf2T, bf2, n):
    steps = n // BN
    g = NUM_GRAPHS

    def body(ag_ref, r2_ref, b_ref, b2_ref, w1_ref, bf1_ref, w2_ref, bf2_ref,
             out_ref, pooled, cnt):
        i = pl.program_id(0)

        @pl.when(i == 0)
        def _():
            pooled[...] = jnp.zeros_like(pooled)
            cnt[...] = jnp.zeros_like(cnt)

        x2 = jnp.concatenate([ag_ref[0], ag_ref[1]], axis=1)
        x2 = jnp.maximum(x2 + r2_ref[...] + b2_ref[...], 0.0)
        bf = b_ref[0]
        gid = lax.broadcasted_iota(jnp.int32, (BN, g), 1).astype(jnp.float32)
        oh = (bf == gid)
        oh = oh.astype(jnp.float32)
        pooled[...] += lax.dot_general(
            oh, x2, (((0,), (0,)), ((), ())),
            preferred_element_type=jnp.float32)
        cnt[...] += lax.dot_general(
            oh, jnp.ones((BN, 64), jnp.float32), (((0,), (0,)), ((), ())),
            preferred_element_type=jnp.float32)

        @pl.when(i == steps - 1)
        def _():
            m = pooled[...] / jnp.maximum(cnt[...], 1.0)
            h1 = jnp.dot(m, w1_ref[...], preferred_element_type=jnp.float32)
            h1 = jnp.maximum(h1 + bf1_ref[...], 0.0)
            out_ref[...] = (jnp.dot(h1, w2_ref[...],
                                    preferred_element_type=jnp.float32)
                            + bf2_ref[...])

    return pl.pallas_call(
        body,
        grid=(steps,),
        in_specs=[
            pl.BlockSpec((2, BN, 32), lambda i: (0, i, 0)),
            pl.BlockSpec((BN, 64), lambda i: (i, 0)),
            pl.BlockSpec((1, BN, 1), lambda i: (i, 0, 0)),
            pl.BlockSpec((1, 64), lambda i: (0, 0)),
            pl.BlockSpec((64, 32), lambda i: (0, 0)),
            pl.BlockSpec((1, 32), lambda i: (0, 0)),
            pl.BlockSpec((32, 3), lambda i: (0, 0)),
            pl.BlockSpec((1, 3), lambda i: (0, 0)),
        ],
        out_specs=pl.BlockSpec((g, 3), lambda i: (0, 0)),
        out_shape=jax.ShapeDtypeStruct((g, 3), jnp.float32),
        scratch_shapes=[
            pltpu.VMEM((g, 64), jnp.float32),
            pltpu.VMEM((g, 64), jnp.float32),
        ],
    )(agg2, r2, batchf, b2, wf1T, bf1, wf2T, bf2)


def kernel(pos, z, edge_index, edge_attr, batch, W1_rel, b1_rel, W1_root,
           W2_rel, b2_rel, W2_root, Wfc1, bfc1, Wfc2, bfc2):
    n = pos.shape[0]
    e = edge_attr.shape[0]
    epad = ((e + 32 * KCH - 1) // (32 * KCH)) * (32 * KCH)
    pad = epad - e
    src = jnp.concatenate([edge_index[0], jnp.zeros((pad,), jnp.int32)])
    dst = jnp.concatenate([edge_index[1], jnp.zeros((pad,), jnp.int32)])
    ew = jnp.concatenate([edge_attr, jnp.zeros((pad,), jnp.float32)])
    src2 = src.reshape(-1, SUB)
    dst2 = dst.reshape(-1, SUB)

    x = jnp.concatenate([pos, z[:, None]], axis=1)
    x16 = jnp.concatenate([x, jnp.zeros((n, 12), jnp.float32)], axis=1)
    npad = ((n + 127) // 128) * 128
    zeros16 = jnp.zeros((npad, 16), jnp.float32)
    zeros32 = jnp.zeros((npad, 32), jnp.float32)

    agg1p = _sc_edge_pass1(src2, dst2, ew, x16, zeros16, n, npad, epad)
    agg1p = agg1p.reshape(2, n, 16)

    ypack, r2 = _tc_mid(agg1p, x, W1_rel.T, W1_root.T, b1_rel[None, :],
                        W2_rel.T, W2_root.T, n)

    agg2 = _sc_edge_pass2(src2, dst2, ew, ypack.reshape(2 * n, 32), zeros32,
                          n, npad, epad)
    agg2 = agg2.reshape(2, n, 32)

    steps = n // BN
    batchf = batch.astype(jnp.float32).reshape(steps, BN, 1)
    out = _tc_final(agg2, r2, batchf, b2_rel[None, :], Wfc1.T, bfc1[None, :],
                    Wfc2.T, bfc2[None, :], n)
    return out


# speedup vs baseline: 13.4651x; 1.2692x over previous
import functools

import jax
import jax.numpy as jnp
from jax import lax
from jax.experimental import pallas as pl
from jax.experimental.pallas import tpu as pltpu
from jax.experimental.pallas import tpu_sc as plsc

SUB = 128
K = 2
KCH = K * SUB
NUM_GRAPHS = 32
BN = 1000


def _scale_rows16(rows, ev, ngroups):
    def scale(q, cc):
        evec = ev[pl.ds(16 * q, 16)]
        for j in range(16):
            r = 16 * q + j
            rows[r] = rows[r] * evec[j]
        return cc

    lax.fori_loop(0, ngroups, scale, 0)


def _scale_rows32(rows, ev, ngroups):
    def scale(q, cc):
        evec = ev[pl.ds(16 * q, 16)]
        for j in range(16):
            r = 16 * q + j
            rows[r, pl.ds(0, 16)] = rows[r, pl.ds(0, 16)] * evec[j]
            rows[r, pl.ds(16, 16)] = rows[r, pl.ds(16, 16)] * evec[j]
        return cc

    lax.fori_loop(0, ngroups, scale, 0)


def _sc_edge_pass1(src2, dst2, ewp, x16, zeros16, n, npad, ep):
    nsk = ep // (32 * KCH)
    assert nsk % 2 == 0
    nrows = npad // 16
    tail = n - 15 * nrows
    mesh = plsc.VectorSubcoreMesh(core_axis_name="c", subcore_axis_name="s")

    @functools.partial(
        pl.kernel, mesh=mesh,
        compiler_params=pltpu.CompilerParams(use_tc_tiling_on_sc=False),
        out_type=jax.ShapeDtypeStruct((2 * n, 16), jnp.float32),
        scratch_types=[
            pltpu.VMEM_SHARED((npad, 16), jnp.float32),
            pltpu.VMEM((K, SUB), jnp.int32),
            pltpu.VMEM((K, SUB), jnp.int32),
            pltpu.VMEM((K, SUB), jnp.int32),
            pltpu.VMEM((K, SUB), jnp.int32),
            pltpu.VMEM((KCH,), jnp.float32),
            pltpu.VMEM((KCH,), jnp.float32),
            pltpu.VMEM((KCH, 16), jnp.float32),
            pltpu.VMEM((KCH, 16), jnp.float32),
            pltpu.SemaphoreType.DMA,
            pltpu.SemaphoreType.DMA,
            pltpu.SemaphoreType.DMA,
            pltpu.SemaphoreType.DMA,
            pltpu.SemaphoreType.DMA,
            pltpu.SemaphoreType.DMA,
            pltpu.SemaphoreType.DMA,
            pltpu.SemaphoreType.DMA,
        ])
    def kern(src_h, dst_h, e_h, x_h, z_h, out_h,
             acc, srcv0, srcv1, dstv0, dstv1, ev0, ev1, rows0, rows1,
             se0, se1, sd0, sd1, sg0, sg1, ss0, ss1):
        c = lax.axis_index("c")
        s = lax.axis_index("s")
        w = s * 2 + c
        r0 = s * nrows
        srcv = (srcv0, srcv1)
        dstv = (dstv0, dstv1)
        ev = (ev0, ev1)
        rows = (rows0, rows1)
        sem_se = (se0, se1)
        sem_d = (sd0, sd1)
        sem_g = (sg0, sg1)
        sem_s = (ss0, ss1)
        gc0 = w * nsk

        def fire_se(gc, b):
            pltpu.async_copy(src_h.at[pl.ds(gc * K, K)], srcv[b], sem_se[b])
            pltpu.async_copy(e_h.at[pl.ds(gc * KCH, KCH)], ev[b], sem_se[b])

        def drain_se(b):
            pltpu.make_async_copy(src_h.at[pl.ds(0, K)], srcv[b],
                                  sem_se[b]).wait()
            pltpu.make_async_copy(e_h.at[pl.ds(0, KCH)], ev[b],
                                  sem_se[b]).wait()

        def fire_d(gc, b):
            pltpu.async_copy(dst_h.at[pl.ds(gc * K, K)], dstv[b], sem_d[b])

        def drain_d(b):
            pltpu.make_async_copy(dst_h.at[pl.ds(0, K)], dstv[b],
                                  sem_d[b]).wait()

        def fire_g(b):
            for j in range(K):
                pltpu.async_copy(x_h.at[srcv[b].at[j]],
                                 rows[b].at[pl.ds(SUB * j, SUB)], sem_g[b])

        def drain_g(b):
            for j in range(K):
                pltpu.make_async_copy(x_h.at[srcv[b].at[j]],
                                      rows[b].at[pl.ds(SUB * j, SUB)],
                                      sem_g[b]).wait()

        def fire_s(b):
            for j in range(K):
                pltpu.async_copy(rows[b].at[pl.ds(SUB * j, SUB)],
                                 acc.at[dstv[b].at[j]], sem_s[b], add=True)

        def drain_s(b):
            for j in range(K):
                pltpu.make_async_copy(rows[b].at[pl.ds(SUB * j, SUB)],
                                      acc.at[dstv[b].at[j]], sem_s[b]).wait()

        pltpu.sync_copy(z_h.at[pl.ds(r0, nrows)], acc.at[pl.ds(r0, nrows)])
        plsc.subcore_barrier()

        fire_se(gc0, 0)
        fire_se(gc0 + 1, 1)
        fire_d(gc0, 0)
        drain_se(0)
        fire_g(0)

        def body(gg, carry):
            for b in (0, 1):
                i = 2 * gg + b
                nb = 1 - b

                @pl.when(i >= 1)
                def _():
                    drain_s(nb)

                @pl.when(i + 1 < nsk)
                def _():
                    fire_d(gc0 + i + 1, nb)
                    drain_se(nb)
                    fire_g(nb)

                drain_d(b)
                drain_g(b)
                _scale_rows16(rows[b], ev[b], KCH // 16)
                fire_s(b)

                @pl.when(i + 2 < nsk)
                def _():
                    fire_se(gc0 + i + 2, b)

            return carry

        lax.fori_loop(0, nsk // 2, body, 0)
        drain_s(1)
        plsc.subcore_barrier()

        @pl.when(s < 15)
        def _():
            pltpu.sync_copy(acc.at[pl.ds(r0, nrows)],
                            out_h.at[pl.ds(c * n + r0, nrows)])

        @pl.when(s == 15)
        def _():
            pltpu.sync_copy(acc.at[pl.ds(r0, tail)],
                            out_h.at[pl.ds(c * n + r0, tail)])

    return kern(src2, dst2, ewp, x16, zeros16)


def _sc_edge_pass2(src2, dst2, ewp, ypack, zeros32, n, npad, ep):
    nsk = ep // (16 * KCH)
    assert nsk % 2 == 0
    nrows = npad // 16
    tail = n - 15 * nrows
    mesh = plsc.VectorSubcoreMesh(core_axis_name="c", subcore_axis_name="s")

    @functools.partial(
        pl.kernel, mesh=mesh,
        compiler_params=pltpu.CompilerParams(use_tc_tiling_on_sc=False),
        out_type=jax.ShapeDtypeStruct((2 * n, 32), jnp.float32),
        scratch_types=[
            pltpu.VMEM_SHARED((npad, 32), jnp.float32),
            pltpu.VMEM((K, SUB), jnp.int32),
            pltpu.VMEM((K, SUB), jnp.int32),
            pltpu.VMEM((K, SUB), jnp.int32),
            pltpu.VMEM((K, SUB), jnp.int32),
            pltpu.VMEM((KCH,), jnp.float32),
            pltpu.VMEM((KCH,), jnp.float32),
            pltpu.VMEM((KCH,), jnp.int32),
            pltpu.VMEM((KCH,), jnp.int32),
            pltpu.VMEM((KCH, 32), jnp.float32),
            pltpu.VMEM((KCH, 32), jnp.float32),
            pltpu.SemaphoreType.DMA,
            pltpu.SemaphoreType.DMA,
            pltpu.SemaphoreType.DMA,
            pltpu.SemaphoreType.DMA,
            pltpu.SemaphoreType.DMA,
            pltpu.SemaphoreType.DMA,
            pltpu.SemaphoreType.DMA,
            pltpu.SemaphoreType.DMA,
        ])
    def kern(src_h, dst_h, e_h, y_h, z_h, out_h,
             acc, srcv0, srcv1, dstv0, dstv1, ev0, ev1, idx20, idx21,
             rows0, rows1, se0, se1, sd0, sd1, sg0, sg1, ss0, ss1):
        c = lax.axis_index("c")
        s = lax.axis_index("s")
        r0 = s * nrows
        coff = c * n
        srcv = (srcv0, srcv1)
        dstv = (dstv0, dstv1)
        ev = (ev0, ev1)
        idx2 = (idx20, idx21)
        rows = (rows0, rows1)
        sem_se = (se0, se1)
        sem_d = (sd0, sd1)
        sem_g = (sg0, sg1)
        sem_s = (ss0, ss1)
        gc0 = s * nsk

        def fire_se(gc, b):
            pltpu.async_copy(src_h.at[pl.ds(gc * K, K)], srcv[b], sem_se[b])
            pltpu.async_copy(e_h.at[pl.ds(gc * KCH, KCH)], ev[b], sem_se[b])

        def drain_se(b):
            pltpu.make_async_copy(src_h.at[pl.ds(0, K)], srcv[b],
                                  sem_se[b]).wait()
            pltpu.make_async_copy(e_h.at[pl.ds(0, KCH)], ev[b],
                                  sem_se[b]).wait()

        def fire_d(gc, b):
            pltpu.async_copy(dst_h.at[pl.ds(gc * K, K)], dstv[b], sem_d[b])

        def drain_d(b):
            pltpu.make_async_copy(dst_h.at[pl.ds(0, K)], dstv[b],
                                  sem_d[b]).wait()

        def idx_add(b):
            for j in range(K):
                for m in range(SUB // 16):
                    idx2[b][pl.ds(SUB * j + 16 * m, 16)] = (
                        srcv[b][j, pl.ds(16 * m, 16)] + coff)

        def fire_g(b):
            for j in range(K):
                pltpu.async_copy(y_h.at[idx2[b].at[pl.ds(SUB * j, SUB)]],
                                 rows[b].at[pl.ds(SUB * j, SUB)], sem_g[b])

        def drain_g(b):
            for j in range(K):
                pltpu.make_async_copy(y_h.at[idx2[b].at[pl.ds(SUB * j, SUB)]],
                                      rows[b].at[pl.ds(SUB * j, SUB)],
                                      sem_g[b]).wait()

        def fire_s(b):
            for j in range(K):
                pltpu.async_copy(rows[b].at[pl.ds(SUB * j, SUB)],
                                 acc.at[dstv[b].at[j]], sem_s[b], add=True)

        def drain_s(b):
            for j in range(K):
                pltpu.make_async_copy(rows[b].at[pl.ds(SUB * j, SUB)],
                                      acc.at[dstv[b].at[j]], sem_s[b]).wait()

        pltpu.sync_copy(z_h.at[pl.ds(r0, nrows)], acc.at[pl.ds(r0, nrows)])
        plsc.subcore_barrier()

        fire_se(gc0, 0)
        fire_se(gc0 + 1, 1)
        fire_d(gc0, 0)
        drain_se(0)
        idx_add(0)
        fire_g(0)

        def body(gg, carry):
            for b in (0, 1):
                i = 2 * gg + b
                nb = 1 - b

                @pl.when(i >= 1)
                def _():
                    drain_s(nb)

                @pl.when(i + 1 < nsk)
                def _():
                    fire_d(gc0 + i + 1, nb)
                    drain_se(nb)
                    idx_add(nb)
                    fire_g(nb)

                drain_d(b)
                drain_g(b)
                _scale_rows32(rows[b], ev[b], KCH // 16)
                fire_s(b)

                @pl.when(i + 2 < nsk)
                def _():
                    fire_se(gc0 + i + 2, b)

            return carry

        lax.fori_loop(0, nsk // 2, body, 0)
        drain_s(1)
        plsc.subcore_barrier()

        @pl.when(s < 15)
        def _():
            pltpu.sync_copy(acc.at[pl.ds(r0, nrows)],
                            out_h.at[pl.ds(coff + r0, nrows)])

        @pl.when(s == 15)
        def _():
            pltpu.sync_copy(acc.at[pl.ds(r0, tail)],
                            out_h.at[pl.ds(coff + r0, tail)])

    return kern(src2, dst2, ewp, ypack, zeros32)


def _tc_mid(agg1p, x, w1relT, w1rootT, b1, w2relT, w2rootT, n):
    steps = n // BN

    def body(ag_ref, x_ref, wa_ref, wb_ref, b1_ref, w2a_ref, w2b_ref,
             yp_ref, r2_ref):
        a = ag_ref[0, :, :4] + ag_ref[1, :, :4]
        x1 = jnp.dot(a, wa_ref[...], preferred_element_type=jnp.float32)
        x1 = x1 + jnp.dot(x_ref[...], wb_ref[...],
                          preferred_element_type=jnp.float32)
        x1 = jnp.maximum(x1 + b1_ref[...], 0.0)
        y = jnp.dot(x1, w2a_ref[...], preferred_element_type=jnp.float32)
        r2_ref[...] = jnp.dot(x1, w2b_ref[...],
                              preferred_element_type=jnp.float32)
        yp_ref[0] = y[:, :32]
        yp_ref[1] = y[:, 32:]

    return pl.pallas_call(
        body,
        grid=(steps,),
        in_specs=[
            pl.BlockSpec((2, BN, 16), lambda i: (0, i, 0)),
            pl.BlockSpec((BN, 4), lambda i: (i, 0)),
            pl.BlockSpec((4, 128), lambda i: (0, 0)),
            pl.BlockSpec((4, 128), lambda i: (0, 0)),
            pl.BlockSpec((1, 128), lambda i: (0, 0)),
            pl.BlockSpec((128, 64), lambda i: (0, 0)),
            pl.BlockSpec((128, 64), lambda i: (0, 0)),
        ],
        out_specs=[
            pl.BlockSpec((2, BN, 32), lambda i: (0, i, 0)),
            pl.BlockSpec((BN, 64), lambda i: (i, 0)),
        ],
        out_shape=[
            jax.ShapeDtypeStruct((2, n, 32), jnp.float32),
            jax.ShapeDtypeStruct((n, 64), jnp.float32),
        ],
    )(agg1p, x, w1relT, w1rootT, b1, w2relT, w2rootT)


def _tc_final(agg2, r2, batchf, b2, wf1T, bf1, wf2T, bf2, n):
    steps = n // BN
    g = NUM_GRAPHS

    def body(ag_ref, r2_ref, b_ref, b2_ref, w1_ref, bf1_ref, w2_ref, bf2_ref,
             out_ref, pooled, cnt):
        i = pl.program_id(0)

        @pl.when(i == 0)
        def _():
            pooled[...] = jnp.zeros_like(pooled)
            cnt[...] = jnp.zeros_like(cnt)

        x2 = jnp.concatenate([ag_ref[0], ag_ref[1]], axis=1)
        x2 = jnp.maximum(x2 + r2_ref[...] + b2_ref[...], 0.0)
        bf = b_ref[0]
        gid = lax.broadcasted_iota(jnp.int32, (BN, g), 1).astype(jnp.float32)
        oh = (bf == gid)
        oh = oh.astype(jnp.float32)
        pooled[...] += lax.dot_general(
            oh, x2, (((0,), (0,)), ((), ())),
            preferred_element_type=jnp.float32)
        cnt[...] += lax.dot_general(
            oh, jnp.ones((BN, 64), jnp.float32), (((0,), (0,)), ((), ())),
            preferred_element_type=jnp.float32)

        @pl.when(i == steps - 1)
        def _():
            m = pooled[...] / jnp.maximum(cnt[...], 1.0)
            h1 = jnp.dot(m, w1_ref[...], preferred_element_type=jnp.float32)
            h1 = jnp.maximum(h1 + bf1_ref[...], 0.0)
            out_ref[...] = (jnp.dot(h1, w2_ref[...],
                                    preferred_element_type=jnp.float32)
                            + bf2_ref[...])

    return pl.pallas_call(
        body,
        grid=(steps,),
        in_specs=[
            pl.BlockSpec((2, BN, 32), lambda i: (0, i, 0)),
            pl.BlockSpec((BN, 64), lambda i: (i, 0)),
            pl.BlockSpec((1, BN, 1), lambda i: (i, 0, 0)),
            pl.BlockSpec((1, 64), lambda i: (0, 0)),
            pl.BlockSpec((64, 32), lambda i: (0, 0)),
            pl.BlockSpec((1, 32), lambda i: (0, 0)),
            pl.BlockSpec((32, 3), lambda i: (0, 0)),
            pl.BlockSpec((1, 3), lambda i: (0, 0)),
        ],
        out_specs=pl.BlockSpec((g, 3), lambda i: (0, 0)),
        out_shape=jax.ShapeDtypeStruct((g, 3), jnp.float32),
        scratch_shapes=[
            pltpu.VMEM((g, 64), jnp.float32),
            pltpu.VMEM((g, 64), jnp.float32),
        ],
    )(agg2, r2, batchf, b2, wf1T, bf1, wf2T, bf2)


def kernel(pos, z, edge_index, edge_attr, batch, W1_rel, b1_rel, W1_root,
           W2_rel, b2_rel, W2_root, Wfc1, bfc1, Wfc2, bfc2):
    n = pos.shape[0]
    e = edge_attr.shape[0]
    epad = ((e + 32 * KCH - 1) // (32 * KCH)) * (32 * KCH)
    pad = epad - e
    src = jnp.concatenate([edge_index[0], jnp.zeros((pad,), jnp.int32)])
    dst = jnp.concatenate([edge_index[1], jnp.zeros((pad,), jnp.int32)])
    ew = jnp.concatenate([edge_attr, jnp.zeros((pad,), jnp.float32)])
    src2 = src.reshape(-1, SUB)
    dst2 = dst.reshape(-1, SUB)

    x = jnp.concatenate([pos, z[:, None]], axis=1)
    x16 = jnp.concatenate([x, jnp.zeros((n, 12), jnp.float32)], axis=1)
    npad = ((n + 127) // 128) * 128
    zeros16 = jnp.zeros((npad, 16), jnp.float32)
    zeros32 = jnp.zeros((npad, 32), jnp.float32)

    agg1p = _sc_edge_pass1(src2, dst2, ew, x16, zeros16, n, npad, epad)
    agg1p = agg1p.reshape(2, n, 16)

    ypack, r2 = _tc_mid(agg1p, x, W1_rel.T, W1_root.T, b1_rel[None, :],
                        W2_rel.T, W2_root.T, n)

    agg2 = _sc_edge_pass2(src2, dst2, ew, ypack.reshape(2 * n, 32), zeros32,
                          n, npad, epad)
    agg2 = agg2.reshape(2, n, 32)

    steps = n // BN
    batchf = batch.astype(jnp.float32).reshape(steps, BN, 1)
    out = _tc_final(agg2, r2, batchf, b2_rel[None, :], Wfc1.T, bfc1[None, :],
                    Wfc2.T, bfc2[None, :], n)
    return out


# speedup vs baseline: 13.6315x; 1.0124x over previous
import functools

import jax
import jax.numpy as jnp
from jax import lax
from jax.experimental import pallas as pl
from jax.experimental.pallas import tpu as pltpu
from jax.experimental.pallas import tpu_sc as plsc

SUB = 128
K = 2
KCH = K * SUB
NUM_GRAPHS = 32
BN = 1000


def _scale_rows16(rows, ev, ngroups):
    for q in range(ngroups):
        evec = ev[pl.ds(16 * q, 16)]
        for j in range(16):
            r = 16 * q + j
            rows[r] = rows[r] * evec[j]


def _scale_rows32(rows, ev, ngroups):
    for q in range(ngroups):
        evec = ev[pl.ds(16 * q, 16)]
        for j in range(16):
            r = 16 * q + j
            rows[r, pl.ds(0, 16)] = rows[r, pl.ds(0, 16)] * evec[j]
            rows[r, pl.ds(16, 16)] = rows[r, pl.ds(16, 16)] * evec[j]


def _sc_edge_pass1(src2, dst2, ewp, x16, zeros16, n, npad, ep):
    nsk = ep // (32 * KCH)
    assert nsk % 2 == 0
    nrows = npad // 16
    tail = n - 15 * nrows
    mesh = plsc.VectorSubcoreMesh(core_axis_name="c", subcore_axis_name="s")

    @functools.partial(
        pl.kernel, mesh=mesh,
        compiler_params=pltpu.CompilerParams(use_tc_tiling_on_sc=False),
        out_type=jax.ShapeDtypeStruct((2 * n, 16), jnp.float32),
        scratch_types=[
            pltpu.VMEM_SHARED((npad, 16), jnp.float32),
            pltpu.VMEM((K, SUB), jnp.int32),
            pltpu.VMEM((K, SUB), jnp.int32),
            pltpu.VMEM((K, SUB), jnp.int32),
            pltpu.VMEM((K, SUB), jnp.int32),
            pltpu.VMEM((KCH,), jnp.float32),
            pltpu.VMEM((KCH,), jnp.float32),
            pltpu.VMEM((KCH, 16), jnp.float32),
            pltpu.VMEM((KCH, 16), jnp.float32),
            pltpu.SemaphoreType.DMA,
            pltpu.SemaphoreType.DMA,
            pltpu.SemaphoreType.DMA,
            pltpu.SemaphoreType.DMA,
            pltpu.SemaphoreType.DMA,
            pltpu.SemaphoreType.DMA,
            pltpu.SemaphoreType.DMA,
            pltpu.SemaphoreType.DMA,
        ])
    def kern(src_h, dst_h, e_h, x_h, z_h, out_h,
             acc, srcv0, srcv1, dstv0, dstv1, ev0, ev1, rows0, rows1,
             se0, se1, sd0, sd1, sg0, sg1, ss0, ss1):
        c = lax.axis_index("c")
        s = lax.axis_index("s")
        w = s * 2 + c
        r0 = s * nrows
        srcv = (srcv0, srcv1)
        dstv = (dstv0, dstv1)
        ev = (ev0, ev1)
        rows = (rows0, rows1)
        sem_se = (se0, se1)
        sem_d = (sd0, sd1)
        sem_g = (sg0, sg1)
        sem_s = (ss0, ss1)
        gc0 = w * nsk

        def fire_se(gc, b):
            pltpu.async_copy(src_h.at[pl.ds(gc * K, K)], srcv[b], sem_se[b])
            pltpu.async_copy(e_h.at[pl.ds(gc * KCH, KCH)], ev[b], sem_se[b])

        def drain_se(b):
            pltpu.make_async_copy(src_h.at[pl.ds(0, K)], srcv[b],
                                  sem_se[b]).wait()
            pltpu.make_async_copy(e_h.at[pl.ds(0, KCH)], ev[b],
                                  sem_se[b]).wait()

        def fire_d(gc, b):
            pltpu.async_copy(dst_h.at[pl.ds(gc * K, K)], dstv[b], sem_d[b])

        def drain_d(b):
            pltpu.make_async_copy(dst_h.at[pl.ds(0, K)], dstv[b],
                                  sem_d[b]).wait()

        def fire_g(b):
            for j in range(K):
                pltpu.async_copy(x_h.at[srcv[b].at[j]],
                                 rows[b].at[pl.ds(SUB * j, SUB)], sem_g[b])

        def drain_g(b):
            for j in range(K):
                pltpu.make_async_copy(x_h.at[srcv[b].at[j]],
                                      rows[b].at[pl.ds(SUB * j, SUB)],
                                      sem_g[b]).wait()

        def fire_s(b):
            for j in range(K):
                pltpu.async_copy(rows[b].at[pl.ds(SUB * j, SUB)],
                                 acc.at[dstv[b].at[j]], sem_s[b], add=True)

        def drain_s(b):
            for j in range(K):
                pltpu.make_async_copy(rows[b].at[pl.ds(SUB * j, SUB)],
                                      acc.at[dstv[b].at[j]], sem_s[b]).wait()

        pltpu.sync_copy(z_h.at[pl.ds(r0, nrows)], acc.at[pl.ds(r0, nrows)])
        plsc.subcore_barrier()

        fire_se(gc0, 0)
        fire_se(gc0 + 1, 1)
        fire_d(gc0, 0)
        drain_se(0)
        fire_g(0)

        def body(gg, carry):
            for b in (0, 1):
                i = 2 * gg + b
                nb = 1 - b

                @pl.when(i >= 1)
                def _():
                    drain_s(nb)

                @pl.when(i + 1 < nsk)
                def _():
                    fire_d(gc0 + i + 1, nb)
                    drain_se(nb)
                    fire_g(nb)

                drain_d(b)
                drain_g(b)
                _scale_rows16(rows[b], ev[b], KCH // 16)
                fire_s(b)

                @pl.when(i + 2 < nsk)
                def _():
                    fire_se(gc0 + i + 2, b)

            return carry

        lax.fori_loop(0, nsk // 2, body, 0)
        drain_s(1)
        plsc.subcore_barrier()

        @pl.when(s < 15)
        def _():
            pltpu.sync_copy(acc.at[pl.ds(r0, nrows)],
                            out_h.at[pl.ds(c * n + r0, nrows)])

        @pl.when(s == 15)
        def _():
            pltpu.sync_copy(acc.at[pl.ds(r0, tail)],
                            out_h.at[pl.ds(c * n + r0, tail)])

    return kern(src2, dst2, ewp, x16, zeros16)


def _sc_edge_pass2(src2, dst2, ewp, ypack, zeros32, n, npad, ep):
    nsk = ep // (16 * KCH)
    assert nsk % 2 == 0
    nrows = npad // 16
    tail = n - 15 * nrows
    mesh = plsc.VectorSubcoreMesh(core_axis_name="c", subcore_axis_name="s")

    @functools.partial(
        pl.kernel, mesh=mesh,
        compiler_params=pltpu.CompilerParams(use_tc_tiling_on_sc=False),
        out_type=jax.ShapeDtypeStruct((2 * n, 32), jnp.float32),
        scratch_types=[
            pltpu.VMEM_SHARED((npad, 32), jnp.float32),
            pltpu.VMEM((K, SUB), jnp.int32),
            pltpu.VMEM((K, SUB), jnp.int32),
            pltpu.VMEM((K, SUB), jnp.int32),
            pltpu.VMEM((K, SUB), jnp.int32),
            pltpu.VMEM((KCH,), jnp.float32),
            pltpu.VMEM((KCH,), jnp.float32),
            pltpu.VMEM((KCH,), jnp.int32),
            pltpu.VMEM((KCH,), jnp.int32),
            pltpu.VMEM((KCH, 32), jnp.float32),
            pltpu.VMEM((KCH, 32), jnp.float32),
            pltpu.SemaphoreType.DMA,
            pltpu.SemaphoreType.DMA,
            pltpu.SemaphoreType.DMA,
            pltpu.SemaphoreType.DMA,
            pltpu.SemaphoreType.DMA,
            pltpu.SemaphoreType.DMA,
            pltpu.SemaphoreType.DMA,
            pltpu.SemaphoreType.DMA,
        ])
    def kern(src_h, dst_h, e_h, y_h, z_h, out_h,
             acc, srcv0, srcv1, dstv0, dstv1, ev0, ev1, idx20, idx21,
             rows0, rows1, se0, se1, sd0, sd1, sg0, sg1, ss0, ss1):
        c = lax.axis_index("c")
        s = lax.axis_index("s")
        r0 = s * nrows
        coff = c * n
        srcv = (srcv0, srcv1)
        dstv = (dstv0, dstv1)
        ev = (ev0, ev1)
        idx2 = (idx20, idx21)
        rows = (rows0, rows1)
        sem_se = (se0, se1)
        sem_d = (sd0, sd1)
        sem_g = (sg0, sg1)
        sem_s = (ss0, ss1)
        gc0 = s * nsk

        def fire_se(gc, b):
            pltpu.async_copy(src_h.at[pl.ds(gc * K, K)], srcv[b], sem_se[b])
            pltpu.async_copy(e_h.at[pl.ds(gc * KCH, KCH)], ev[b], sem_se[b])

        def drain_se(b):
            pltpu.make_async_copy(src_h.at[pl.ds(0, K)], srcv[b],
                                  sem_se[b]).wait()
            pltpu.make_async_copy(e_h.at[pl.ds(0, KCH)], ev[b],
                                  sem_se[b]).wait()

        def fire_d(gc, b):
            pltpu.async_copy(dst_h.at[pl.ds(gc * K, K)], dstv[b], sem_d[b])

        def drain_d(b):
            pltpu.make_async_copy(dst_h.at[pl.ds(0, K)], dstv[b],
                                  sem_d[b]).wait()

        def idx_add(b):
            for j in range(K):
                for m in range(SUB // 16):
                    idx2[b][pl.ds(SUB * j + 16 * m, 16)] = (
                        srcv[b][j, pl.ds(16 * m, 16)] + coff)

        def fire_g(b):
            for j in range(K):
                pltpu.async_copy(y_h.at[idx2[b].at[pl.ds(SUB * j, SUB)]],
                                 rows[b].at[pl.ds(SUB * j, SUB)], sem_g[b])

        def drain_g(b):
            for j in range(K):
                pltpu.make_async_copy(y_h.at[idx2[b].at[pl.ds(SUB * j, SUB)]],
                                      rows[b].at[pl.ds(SUB * j, SUB)],
                                      sem_g[b]).wait()

        def fire_s(b):
            for j in range(K):
                pltpu.async_copy(rows[b].at[pl.ds(SUB * j, SUB)],
                                 acc.at[dstv[b].at[j]], sem_s[b], add=True)

        def drain_s(b):
            for j in range(K):
                pltpu.make_async_copy(rows[b].at[pl.ds(SUB * j, SUB)],
                                      acc.at[dstv[b].at[j]], sem_s[b]).wait()

        pltpu.sync_copy(z_h.at[pl.ds(r0, nrows)], acc.at[pl.ds(r0, nrows)])
        plsc.subcore_barrier()

        fire_se(gc0, 0)
        fire_se(gc0 + 1, 1)
        fire_d(gc0, 0)
        drain_se(0)
        idx_add(0)
        fire_g(0)

        def body(gg, carry):
            for b in (0, 1):
                i = 2 * gg + b
                nb = 1 - b

                @pl.when(i >= 1)
                def _():
                    drain_s(nb)

                @pl.when(i + 1 < nsk)
                def _():
                    fire_d(gc0 + i + 1, nb)
                    drain_se(nb)
                    idx_add(nb)
                    fire_g(nb)

                drain_d(b)
                drain_g(b)
                _scale_rows32(rows[b], ev[b], KCH // 16)
                fire_s(b)

                @pl.when(i + 2 < nsk)
                def _():
                    fire_se(gc0 + i + 2, b)

            return carry

        lax.fori_loop(0, nsk // 2, body, 0)
        drain_s(1)
        plsc.subcore_barrier()

        @pl.when(s < 15)
        def _():
            pltpu.sync_copy(acc.at[pl.ds(r0, nrows)],
                            out_h.at[pl.ds(coff + r0, nrows)])

        @pl.when(s == 15)
        def _():
            pltpu.sync_copy(acc.at[pl.ds(r0, tail)],
                            out_h.at[pl.ds(coff + r0, tail)])

    return kern(src2, dst2, ewp, ypack, zeros32)


def _tc_mid(agg1p, x, w1relT, w1rootT, b1, w2relT, w2rootT, n):
    steps = n // BN

    def body(ag_ref, x_ref, wa_ref, wb_ref, b1_ref, w2a_ref, w2b_ref,
             yp_ref, r2_ref):
        a = ag_ref[0, :, :4] + ag_ref[1, :, :4]
        x1 = jnp.dot(a, wa_ref[...], preferred_element_type=jnp.float32)
        x1 = x1 + jnp.dot(x_ref[...], wb_ref[...],
                          preferred_element_type=jnp.float32)
        x1 = jnp.maximum(x1 + b1_ref[...], 0.0)
        y = jnp.dot(x1, w2a_ref[...], preferred_element_type=jnp.float32)
        r2_ref[...] = jnp.dot(x1, w2b_ref[...],
                              preferred_element_type=jnp.float32)
        yp_ref[0] = y[:, :32]
        yp_ref[1] = y[:, 32:]

    return pl.pallas_call(
        body,
        grid=(steps,),
        in_specs=[
            pl.BlockSpec((2, BN, 16), lambda i: (0, i, 0)),
            pl.BlockSpec((BN, 4), lambda i: (i, 0)),
            pl.BlockSpec((4, 128), lambda i: (0, 0)),
            pl.BlockSpec((4, 128), lambda i: (0, 0)),
            pl.BlockSpec((1, 128), lambda i: (0, 0)),
            pl.BlockSpec((128, 64), lambda i: (0, 0)),
            pl.BlockSpec((128, 64), lambda i: (0, 0)),
        ],
        out_specs=[
            pl.BlockSpec((2, BN, 32), lambda i: (0, i, 0)),
            pl.BlockSpec((BN, 64), lambda i: (i, 0)),
        ],
        out_shape=[
            jax.ShapeDtypeStruct((2, n, 32), jnp.float32),
            jax.ShapeDtypeStruct((n, 64), jnp.float32),
        ],
    )(agg1p, x, w1relT, w1rootT, b1, w2relT, w2rootT)


def _tc_final(agg2, r2, batchf, b2, wf1T, bf1, wf2T, bf2, n):
    steps = n // BN
    g = NUM_GRAPHS

    def body(ag_ref, r2_ref, b_ref, b2_ref, w1_ref, bf1_ref, w2_ref, bf2_ref,
             out_ref, pooled, cnt):
        i = pl.program_id(0)

        @pl.when(i == 0)
        def _():
            pooled[...] = jnp.zeros_like(pooled)
            cnt[...] = jnp.zeros_like(cnt)

        x2 = jnp.concatenate([ag_ref[0], ag_ref[1]], axis=1)
        x2 = jnp.maximum(x2 + r2_ref[...] + b2_ref[...], 0.0)
        bf = b_ref[0]
        gid = lax.broadcasted_iota(jnp.int32, (BN, g), 1).astype(jnp.float32)
        oh = (bf == gid)
        oh = oh.astype(jnp.float32)
        pooled[...] += lax.dot_general(
            oh, x2, (((0,), (0,)), ((), ())),
            preferred_element_type=jnp.float32)
        cnt[...] += lax.dot_general(
            oh, jnp.ones((BN, 64), jnp.float32), (((0,), (0,)), ((), ())),
            preferred_element_type=jnp.float32)

        @pl.when(i == steps - 1)
        def _():
            m = pooled[...] / jnp.maximum(cnt[...], 1.0)
            h1 = jnp.dot(m, w1_ref[...], preferred_element_type=jnp.float32)
            h1 = jnp.maximum(h1 + bf1_ref[...], 0.0)
            out_ref[...] = (jnp.dot(h1, w2_ref[...],
                                    preferred_element_type=jnp.float32)
                            + bf2_ref[...])

    return pl.pallas_call(
        body,
        grid=(steps,),
        in_specs=[
            pl.BlockSpec((2, BN, 32), lambda i: (0, i, 0)),
            pl.BlockSpec((BN, 64), lambda i: (i, 0)),
            pl.BlockSpec((1, BN, 1), lambda i: (i, 0, 0)),
            pl.BlockSpec((1, 64), lambda i: (0, 0)),
            pl.BlockSpec((64, 32), lambda i: (0, 0)),
            pl.BlockSpec((1, 32), lambda i: (0, 0)),
            pl.BlockSpec((32, 3), lambda i: (0, 0)),
            pl.BlockSpec((1, 3), lambda i: (0, 0)),
        ],
        out_specs=pl.BlockSpec((g, 3), lambda i: (0, 0)),
        out_shape=jax.ShapeDtypeStruct((g, 3), jnp.float32),
        scratch_shapes=[
            pltpu.VMEM((g, 64), jnp.float32),
            pltpu.VMEM((g, 64), jnp.float32),
        ],
    )(agg2, r2, batchf, b2, wf1T, bf1, wf2T, bf2)


def kernel(pos, z, edge_index, edge_attr, batch, W1_rel, b1_rel, W1_root,
           W2_rel, b2_rel, W2_root, Wfc1, bfc1, Wfc2, bfc2):
    n = pos.shape[0]
    e = edge_attr.shape[0]
    epad = ((e + 32 * KCH - 1) // (32 * KCH)) * (32 * KCH)
    pad = epad - e
    src = jnp.concatenate([edge_index[0], jnp.zeros((pad,), jnp.int32)])
    dst = jnp.concatenate([edge_index[1], jnp.zeros((pad,), jnp.int32)])
    ew = jnp.concatenate([edge_attr, jnp.zeros((pad,), jnp.float32)])
    src2 = src.reshape(-1, SUB)
    dst2 = dst.reshape(-1, SUB)

    x = jnp.concatenate([pos, z[:, None]], axis=1)
    x16 = jnp.concatenate([x, jnp.zeros((n, 12), jnp.float32)], axis=1)
    npad = ((n + 127) // 128) * 128
    zeros16 = jnp.zeros((npad, 16), jnp.float32)
    zeros32 = jnp.zeros((npad, 32), jnp.float32)

    agg1p = _sc_edge_pass1(src2, dst2, ew, x16, zeros16, n, npad, epad)
    agg1p = agg1p.reshape(2, n, 16)

    ypack, r2 = _tc_mid(agg1p, x, W1_rel.T, W1_root.T, b1_rel[None, :],
                        W2_rel.T, W2_root.T, n)

    agg2 = _sc_edge_pass2(src2, dst2, ew, ypack.reshape(2 * n, 32), zeros32,
                          n, npad, epad)
    agg2 = agg2.reshape(2, n, 32)

    steps = n // BN
    batchf = batch.astype(jnp.float32).reshape(steps, BN, 1)
    out = _tc_final(agg2, r2, batchf, b2_rel[None, :], Wfc1.T, bfc1[None, :],
                    Wfc2.T, bfc2[None, :], n)
    return out
